# baseline (speedup 1.0000x reference)
import functools

import jax
import jax.numpy as jnp
from jax import lax
from jax.experimental import pallas as pl
from jax.experimental.pallas import tpu as pltpu
from jax.experimental.pallas import tpu_sc as plsc

NUM_GRAPHS = 512
N_NODES = 100000
D_NODE = 128
D_U = 64
D_REP = 128

NC = 2
NS = 16
NW = NC * NS
SUB = 80
NSUB = 5
CHUNK = SUB * NSUB
NCHUNKS = N_NODES // CHUNK
MAXCH = -(-NCHUNKS // NW)
CW = 128
SEG_PER_SUB = NUM_GRAPHS // NS


@functools.partial(
    pl.kernel,
    mesh=plsc.VectorSubcoreMesh(core_axis_name="c", subcore_axis_name="s"),
    out_type=[
        jax.ShapeDtypeStruct((NC, NUM_GRAPHS, D_NODE), jnp.float32),
        jax.ShapeDtypeStruct((NC, NUM_GRAPHS, CW), jnp.float32),
    ],
    scratch_types=[
        pltpu.VMEM((2, CHUNK, D_NODE), jnp.float32),
        pltpu.VMEM((2, NSUB, SUB), jnp.int32),
        pltpu.VMEM((SUB, CW), jnp.float32),
        pltpu.VMEM((SEG_PER_SUB, D_NODE), jnp.float32),
        pltpu.VMEM_SHARED((NUM_GRAPHS, D_NODE), jnp.float32),
        pltpu.VMEM_SHARED((NUM_GRAPHS, CW), jnp.float32),
        pltpu.SemaphoreType.DMA((2,)),
        pltpu.SemaphoreType.DMA((2,)),
    ],
)
def _sc_segment_sum(x_hbm, seg_hbm, out_sums, out_cnts,
                    xbuf, idxbuf, onesbuf, zbuf, acc_sh, cnt_sh, xsem, isem):
    c = lax.axis_index("c")
    s = lax.axis_index("s")
    wid = s * NC + c

    def _fill_row(i, _):
        @pl.when(i < SEG_PER_SUB)
        def _():
            for j in range(D_NODE // 16):
                zbuf[i, pl.ds(j * 16, 16)] = jnp.zeros((16,), jnp.float32)
        for j in range(CW // 16):
            onesbuf[i, pl.ds(j * 16, 16)] = jnp.ones((16,), jnp.float32)
        return 0

    lax.fori_loop(0, SUB, _fill_row, 0)

    pltpu.sync_copy(zbuf, acc_sh.at[pl.ds(s * SEG_PER_SUB, SEG_PER_SUB)])
    pltpu.sync_copy(zbuf, cnt_sh.at[pl.ds(s * SEG_PER_SUB, SEG_PER_SUB)])
    plsc.subcore_barrier()

    def _loads(k, b):
        base = k * CHUNK
        pltpu.make_async_copy(x_hbm.at[pl.ds(base, CHUNK)], xbuf.at[b], xsem.at[b]).start()
        pltpu.make_async_copy(seg_hbm.at[k], idxbuf.at[b], isem.at[b]).start()

    def _wait_loads(k, b):
        base = k * CHUNK
        pltpu.make_async_copy(x_hbm.at[pl.ds(base, CHUNK)], xbuf.at[b], xsem.at[b]).wait()
        pltpu.make_async_copy(seg_hbm.at[k], idxbuf.at[b], isem.at[b]).wait()

    _loads(wid, 0)

    def _step(i2, _):
        for b in range(2):
            ii = i2 * 2 + b
            k = wid + ii * NW
            kn = k + NW

            @pl.when(kn < NCHUNKS)
            def _():
                _loads(kn, (b + 1) % 2)

            @pl.when(k < NCHUNKS)
            def _():
                _wait_loads(k, b)
                for j in range(NSUB):
                    pltpu.sync_copy(xbuf.at[b, pl.ds(j * SUB, SUB)],
                                    acc_sh.at[idxbuf.at[b, j]], add=True)
                    pltpu.sync_copy(onesbuf, cnt_sh.at[idxbuf.at[b, j]], add=True)

        return 0

    lax.fori_loop(0, MAXCH // 2, _step, 0)
    plsc.subcore_barrier()

    pltpu.sync_copy(acc_sh.at[pl.ds(s * SEG_PER_SUB, SEG_PER_SUB)],
                    out_sums.at[c, pl.ds(s * SEG_PER_SUB, SEG_PER_SUB)])
    pltpu.sync_copy(cnt_sh.at[pl.ds(s * SEG_PER_SUB, SEG_PER_SUB)],
                    out_cnts.at[c, pl.ds(s * SEG_PER_SUB, SEG_PER_SUB)])


def _mlp_body(sums_ref, cnts_ref, u_ref, w1_ref, b1_ref, g_ref, be_ref,
              w2_ref, b2_ref, out_ref):
    sums = sums_ref[0] + sums_ref[1]
    cnts = cnts_ref[0] + cnts_ref[1]
    cnt = cnts[:, 0:1]
    mean = sums / jnp.maximum(cnt, 1.0)
    h = (jnp.dot(u_ref[...], w1_ref[0:D_U, :], preferred_element_type=jnp.float32)
         + jnp.dot(mean, w1_ref[D_U:, :], preferred_element_type=jnp.float32)
         + b1_ref[...])
    h = jnp.where(h >= 0, h, 0.01 * h)
    mu = jnp.mean(h, axis=-1, keepdims=True)
    var = jnp.mean((h - mu) ** 2, axis=-1, keepdims=True)
    h = (h - mu) * lax.rsqrt(var + 1e-5) * g_ref[...] + be_ref[...]
    out_ref[...] = jnp.dot(h, w2_ref[...], preferred_element_type=jnp.float32) + b2_ref[...]


def kernel(x, edge_index, edge_attr, u, batch, W1, b1, gamma, beta, W2, b2):
    del edge_index, edge_attr
    seg = batch.astype(jnp.int32).reshape(NCHUNKS, NSUB, SUB)
    sums2, cnts2 = _sc_segment_sum(x, seg)
    return pl.pallas_call(
        _mlp_body,
        out_shape=jax.ShapeDtypeStruct((NUM_GRAPHS, D_REP), jnp.float32),
    )(sums2, cnts2, u, W1, b1.reshape(1, -1), gamma.reshape(1, -1),
      beta.reshape(1, -1), W2, b2.reshape(1, -1))


# speedup vs baseline: 1.2009x; 1.2009x over previous
import functools

import jax
import jax.numpy as jnp
from jax import lax
from jax.experimental import pallas as pl
from jax.experimental.pallas import tpu as pltpu
from jax.experimental.pallas import tpu_sc as plsc

NUM_GRAPHS = 512
N_NODES = 100000
D_NODE = 128
D_U = 64
D_REP = 128

NC = 2
NS = 16
NW = NC * NS
CHUNK = 128
NFULL = N_NODES // CHUNK
TAIL = N_NODES - NFULL * CHUNK
TAIL_BASE = NFULL * CHUNK
MAXCH = -(-NFULL // NW)
CW = 128
SEG_PER_SUB = NUM_GRAPHS // NS


@functools.partial(
    pl.kernel,
    mesh=plsc.VectorSubcoreMesh(core_axis_name="c", subcore_axis_name="s"),
    out_type=[
        jax.ShapeDtypeStruct((NC, NUM_GRAPHS, D_NODE), jnp.float32),
        jax.ShapeDtypeStruct((NC, NUM_GRAPHS, CW), jnp.float32),
    ],
    scratch_types=[
        pltpu.VMEM((2, CHUNK, D_NODE), jnp.float32),
        pltpu.VMEM((2, 1, CHUNK), jnp.int32),
        pltpu.VMEM((1, TAIL), jnp.int32),
        pltpu.VMEM((CHUNK, CW), jnp.float32),
        pltpu.VMEM((SEG_PER_SUB, D_NODE), jnp.float32),
        pltpu.VMEM_SHARED((NUM_GRAPHS, D_NODE), jnp.float32),
        pltpu.VMEM_SHARED((NUM_GRAPHS, CW), jnp.float32),
        pltpu.SemaphoreType.DMA((2,)),
        pltpu.SemaphoreType.DMA((2,)),
    ],
)
def _sc_segment_sum(x_hbm, seg_hbm, out_sums, out_cnts,
                    xbuf, idxbuf, tidx, onesbuf, zbuf, acc_sh, cnt_sh, xsem, isem):
    c = lax.axis_index("c")
    s = lax.axis_index("s")
    wid = s * NC + c

    def _fill_row(i, _):
        @pl.when(i < SEG_PER_SUB)
        def _():
            for j in range(D_NODE // 16):
                zbuf[i, pl.ds(j * 16, 16)] = jnp.zeros((16,), jnp.float32)
        for j in range(CW // 16):
            onesbuf[i, pl.ds(j * 16, 16)] = jnp.ones((16,), jnp.float32)
        return 0

    lax.fori_loop(0, CHUNK, _fill_row, 0)

    pltpu.sync_copy(zbuf, acc_sh.at[pl.ds(s * SEG_PER_SUB, SEG_PER_SUB)])
    pltpu.sync_copy(zbuf, cnt_sh.at[pl.ds(s * SEG_PER_SUB, SEG_PER_SUB)])
    plsc.subcore_barrier()

    def _loads(k, b):
        base = k * CHUNK
        pltpu.make_async_copy(x_hbm.at[pl.ds(base, CHUNK)], xbuf.at[b], xsem.at[b]).start()
        pltpu.make_async_copy(seg_hbm.at[pl.ds(base, CHUNK)], idxbuf.at[b, 0], isem.at[b]).start()

    def _wait_loads(k, b):
        base = k * CHUNK
        pltpu.make_async_copy(x_hbm.at[pl.ds(base, CHUNK)], xbuf.at[b], xsem.at[b]).wait()
        pltpu.make_async_copy(seg_hbm.at[pl.ds(base, CHUNK)], idxbuf.at[b, 0], isem.at[b]).wait()

    _loads(wid, 0)

    def _step(i2, _):
        for b in range(2):
            ii = i2 * 2 + b
            k = wid + ii * NW
            kn = k + NW

            @pl.when(kn < NFULL)
            def _():
                _loads(kn, (b + 1) % 2)

            @pl.when(k < NFULL)
            def _():
                _wait_loads(k, b)
                pltpu.sync_copy(xbuf.at[b], acc_sh.at[idxbuf.at[b, 0]], add=True)
                pltpu.sync_copy(onesbuf, cnt_sh.at[idxbuf.at[b, 0]], add=True)

        return 0

    lax.fori_loop(0, (MAXCH + 1) // 2, _step, 0)

    @pl.when(wid == NW - 1)
    def _():
        pltpu.sync_copy(x_hbm.at[pl.ds(TAIL_BASE, TAIL)], xbuf.at[0, pl.ds(0, TAIL)])
        pltpu.sync_copy(seg_hbm.at[pl.ds(TAIL_BASE, TAIL)], tidx.at[0])
        pltpu.sync_copy(xbuf.at[0, pl.ds(0, TAIL)], acc_sh.at[tidx.at[0]], add=True)
        pltpu.sync_copy(onesbuf.at[pl.ds(0, TAIL)], cnt_sh.at[tidx.at[0]], add=True)

    plsc.subcore_barrier()

    pltpu.sync_copy(acc_sh.at[pl.ds(s * SEG_PER_SUB, SEG_PER_SUB)],
                    out_sums.at[c, pl.ds(s * SEG_PER_SUB, SEG_PER_SUB)])
    pltpu.sync_copy(cnt_sh.at[pl.ds(s * SEG_PER_SUB, SEG_PER_SUB)],
                    out_cnts.at[c, pl.ds(s * SEG_PER_SUB, SEG_PER_SUB)])


def _mlp_body(sums_ref, cnts_ref, u_ref, w1_ref, b1_ref, g_ref, be_ref,
              w2_ref, b2_ref, out_ref):
    sums = sums_ref[0] + sums_ref[1]
    cnts = cnts_ref[0] + cnts_ref[1]
    cnt = cnts[:, 0:1]
    mean = sums / jnp.maximum(cnt, 1.0)
    h = (jnp.dot(u_ref[...], w1_ref[0:D_U, :], preferred_element_type=jnp.float32)
         + jnp.dot(mean, w1_ref[D_U:, :], preferred_element_type=jnp.float32)
         + b1_ref[...])
    h = jnp.where(h >= 0, h, 0.01 * h)
    mu = jnp.mean(h, axis=-1, keepdims=True)
    var = jnp.mean((h - mu) ** 2, axis=-1, keepdims=True)
    h = (h - mu) * lax.rsqrt(var + 1e-5) * g_ref[...] + be_ref[...]
    out_ref[...] = jnp.dot(h, w2_ref[...], preferred_element_type=jnp.float32) + b2_ref[...]


def kernel(x, edge_index, edge_attr, u, batch, W1, b1, gamma, beta, W2, b2):
    del edge_index, edge_attr
    seg = batch.astype(jnp.int32)
    sums2, cnts2 = _sc_segment_sum(x, seg)
    return pl.pallas_call(
        _mlp_body,
        out_shape=jax.ShapeDtypeStruct((NUM_GRAPHS, D_REP), jnp.float32),
    )(sums2, cnts2, u, W1, b1.reshape(1, -1), gamma.reshape(1, -1),
      beta.reshape(1, -1), W2, b2.reshape(1, -1))


# speedup vs baseline: 1.5456x; 1.2871x over previous
import functools

import jax
import jax.numpy as jnp
from jax import lax
from jax.experimental import pallas as pl
from jax.experimental.pallas import tpu as pltpu
from jax.experimental.pallas import tpu_sc as plsc

NUM_GRAPHS = 512
N_NODES = 100000
D_NODE = 128
D_U = 64
D_REP = 128

NC = 2
NS = 16
NW = NC * NS
CHUNK = 128
NFULL = N_NODES // CHUNK
TAIL = N_NODES - NFULL * CHUNK
TAIL_BASE = NFULL * CHUNK
MAXCH = -(-NFULL // NW)
SEG_PER_SUB = NUM_GRAPHS // NS


@functools.partial(
    pl.kernel,
    mesh=plsc.VectorSubcoreMesh(core_axis_name="c", subcore_axis_name="s"),
    compiler_params=pltpu.CompilerParams(needs_layout_passes=False),
    out_type=[
        jax.ShapeDtypeStruct((NC, NUM_GRAPHS, D_NODE), jnp.float32),
        jax.ShapeDtypeStruct((NW, NUM_GRAPHS), jnp.float32),
    ],
    scratch_types=[
        pltpu.VMEM((2, CHUNK, D_NODE), jnp.float32),
        pltpu.VMEM((2, 1, CHUNK), jnp.int32),
        pltpu.VMEM((1, TAIL), jnp.int32),
        pltpu.VMEM((NUM_GRAPHS,), jnp.float32),
        pltpu.VMEM((SEG_PER_SUB, D_NODE), jnp.float32),
        pltpu.VMEM_SHARED((NUM_GRAPHS, D_NODE), jnp.float32),
        pltpu.SemaphoreType.DMA((2,)),
        pltpu.SemaphoreType.DMA((2,)),
    ],
)
def _sc_segment_sum(x_hbm, seg_hbm, out_sums, out_cnts,
                    xbuf, idxbuf, tidx, cnt_loc, zbuf, acc_sh, xsem, isem):
    c = lax.axis_index("c")
    s = lax.axis_index("s")
    wid = s * NC + c

    def _fill_row(i, _):
        @pl.when(i < SEG_PER_SUB)
        def _():
            for j in range(D_NODE // 16):
                zbuf[i, pl.ds(j * 16, 16)] = jnp.zeros((16,), jnp.float32)
        cnt_loc[pl.ds(i * 16, 16)] = jnp.zeros((16,), jnp.float32)
        return 0

    lax.fori_loop(0, NUM_GRAPHS // 16, _fill_row, 0)

    pltpu.sync_copy(zbuf, acc_sh.at[pl.ds(s * SEG_PER_SUB, SEG_PER_SUB)])
    plsc.subcore_barrier()

    ones16 = jnp.ones((16,), jnp.float32)

    def _count(idx_row, n_vecs):
        for j in range(n_vecs):
            idxv = idx_row[pl.ds(j * 16, 16)]
            plsc.addupdate_scatter(cnt_loc, [idxv], ones16)

    def _loads(k, b):
        base = k * CHUNK
        pltpu.make_async_copy(x_hbm.at[pl.ds(base, CHUNK)], xbuf.at[b], xsem.at[b]).start()
        pltpu.make_async_copy(seg_hbm.at[pl.ds(base, CHUNK)], idxbuf.at[b, 0], isem.at[b]).start()

    def _wait_loads(k, b):
        base = k * CHUNK
        pltpu.make_async_copy(x_hbm.at[pl.ds(base, CHUNK)], xbuf.at[b], xsem.at[b]).wait()
        pltpu.make_async_copy(seg_hbm.at[pl.ds(base, CHUNK)], idxbuf.at[b, 0], isem.at[b]).wait()

    _loads(wid, 0)

    def _step(i2, _):
        for b in range(2):
            ii = i2 * 2 + b
            k = wid + ii * NW
            kn = k + NW

            @pl.when(kn < NFULL)
            def _():
                _loads(kn, (b + 1) % 2)

            @pl.when(k < NFULL)
            def _():
                _wait_loads(k, b)
                pltpu.sync_copy(xbuf.at[b], acc_sh.at[idxbuf.at[b, 0]], add=True)
                _count(idxbuf.at[b, 0], CHUNK // 16)

        return 0

    lax.fori_loop(0, (MAXCH + 1) // 2, _step, 0)

    @pl.when(wid == NW - 1)
    def _():
        pltpu.sync_copy(x_hbm.at[pl.ds(TAIL_BASE, TAIL)], xbuf.at[0, pl.ds(0, TAIL)])
        pltpu.sync_copy(seg_hbm.at[pl.ds(TAIL_BASE, TAIL)], tidx.at[0])
        pltpu.sync_copy(xbuf.at[0, pl.ds(0, TAIL)], acc_sh.at[tidx.at[0]], add=True)
        _count(tidx.at[0], TAIL // 16)

    plsc.subcore_barrier()

    pltpu.sync_copy(acc_sh.at[pl.ds(s * SEG_PER_SUB, SEG_PER_SUB)],
                    out_sums.at[c, pl.ds(s * SEG_PER_SUB, SEG_PER_SUB)])
    pltpu.sync_copy(cnt_loc, out_cnts.at[wid])


def _mlp_body(sums_ref, cnts_ref, u_ref, w1_ref, b1_ref, g_ref, be_ref,
              w2_ref, b2_ref, out_ref):
    sums = sums_ref[0] + sums_ref[1]
    cnt = lax.dot_general(cnts_ref[...], jnp.ones((NW, 1), jnp.float32),
                          (((0,), (0,)), ((), ())),
                          preferred_element_type=jnp.float32)
    mean = sums / jnp.maximum(cnt, 1.0)
    h = (jnp.dot(u_ref[...], w1_ref[0:D_U, :], preferred_element_type=jnp.float32)
         + jnp.dot(mean, w1_ref[D_U:, :], preferred_element_type=jnp.float32)
         + b1_ref[...])
    h = jnp.where(h >= 0, h, 0.01 * h)
    mu = jnp.mean(h, axis=-1, keepdims=True)
    var = jnp.mean((h - mu) ** 2, axis=-1, keepdims=True)
    h = (h - mu) * lax.rsqrt(var + 1e-5) * g_ref[...] + be_ref[...]
    out_ref[...] = jnp.dot(h, w2_ref[...], preferred_element_type=jnp.float32) + b2_ref[...]


def kernel(x, edge_index, edge_attr, u, batch, W1, b1, gamma, beta, W2, b2):
    del edge_index, edge_attr
    seg = batch.astype(jnp.int32)
    sums2, cnts2 = _sc_segment_sum(x, seg)
    return pl.pallas_call(
        _mlp_body,
        out_shape=jax.ShapeDtypeStruct((NUM_GRAPHS, D_REP), jnp.float32),
    )(sums2, cnts2, u, W1, b1.reshape(1, -1), gamma.reshape(1, -1),
      beta.reshape(1, -1), W2, b2.reshape(1, -1))
